# baseline (speedup 1.0000x reference)
import functools

import jax
import jax.numpy as jnp
from jax import lax
from jax.experimental import pallas as pl
from jax.experimental.pallas import tpu as pltpu
from jax.experimental.pallas import tpu_sc as plsc

NP = 10240
EB = 64
NW = 32
EP = 163840


def _sc_edge_agg(A_cm, U_cm, src, dst, Hch):
    Hc = 128
    nb_w = EP // EB // NW
    rows_s = NP // 16
    ZR = 64

    mesh = plsc.VectorSubcoreMesh(core_axis_name="c", subcore_axis_name="s")

    @functools.partial(
        pl.kernel, mesh=mesh,
        out_type=jax.ShapeDtypeStruct((Hch * 2 * NP, Hc), jnp.float32),
        scratch_types=[
            pltpu.VMEM((EB, Hc), jnp.float32),
            pltpu.VMEM((EB, Hc), jnp.float32),
            pltpu.VMEM((EB, Hc), jnp.float32),
            pltpu.VMEM((EB, Hc), jnp.float32),
            pltpu.VMEM((EB,), jnp.int32),
            pltpu.VMEM((EB,), jnp.int32),
            pltpu.VMEM((EB,), jnp.int32),
            pltpu.VMEM((EB,), jnp.int32),
            pltpu.VMEM((EB,), jnp.int32),
            pltpu.VMEM((EB,), jnp.int32),
            pltpu.VMEM((EB,), jnp.int32),
            pltpu.VMEM((EB,), jnp.int32),
            pltpu.VMEM((EB,), jnp.int32),
            pltpu.VMEM((EB,), jnp.int32),
            pltpu.VMEM_SHARED((NP, Hc), jnp.float32),
            pltpu.SemaphoreType.DMA,
            pltpu.SemaphoreType.DMA,
            pltpu.SemaphoreType.DMA,
            pltpu.SemaphoreType.DMA,
            pltpu.SemaphoreType.DMA,
            pltpu.SemaphoreType.DMA,
            pltpu.SemaphoreType.DMA,
            pltpu.SemaphoreType.DMA,
        ],
    )
    def k(a_hbm, u_hbm, src_hbm, dst_hbm, out_hbm,
          a0, u0, a1, u1, is0, id0, iso0, idu0, is1, id1, iso1, idu1,
          idc0, idc1, acc, sa0, su0, sa1, su1, si0, si1, ss0, ss1):
        cid = lax.axis_index("c")
        sid = lax.axis_index("s")
        wid = sid * 2 + cid
        bufs = ((a0, u0, is0, id0, iso0, idu0, sa0, su0, si0, idc0, ss0),
                (a1, u1, is1, id1, iso1, idu1, sa1, su1, si1, idc1, ss1))

        def icp(b, j):
            isj, idj, si = bufs[j][2], bufs[j][3], bufs[j][8]
            base = (wid * nb_w + b) * EB
            pltpu.make_async_copy(src_hbm.at[pl.ds(base, EB)], isj, si).start()
            pltpu.make_async_copy(dst_hbm.at[pl.ds(base, EB)], idj, si).start()

        def icp_wait(b, j):
            isj, idj, si = bufs[j][2], bufs[j][3], bufs[j][8]
            base = (wid * nb_w + b) * EB
            pltpu.make_async_copy(src_hbm.at[pl.ds(base, EB)], isj, si).wait()
            pltpu.make_async_copy(dst_hbm.at[pl.ds(base, EB)], idj, si).wait()

        def off(j, coff):
            isj, idj, isoj, iduj = bufs[j][2], bufs[j][3], bufs[j][4], bufs[j][5]
            for jj in range(EB // 16):
                sl = pl.ds(jj * 16, 16)
                isoj[sl] = isj[sl] + coff
                iduj[sl] = idj[sl] + coff

        def gath(j):
            av, uv, isoj, iduj, sa, su = (bufs[j][0], bufs[j][1], bufs[j][4],
                                          bufs[j][5], bufs[j][6], bufs[j][7])
            pltpu.make_async_copy(a_hbm.at[isoj], av, sa).start()
            pltpu.make_async_copy(u_hbm.at[iduj], uv, su).start()

        def gath_wait(j):
            av, uv, isoj, iduj, sa, su = (bufs[j][0], bufs[j][1], bufs[j][4],
                                          bufs[j][5], bufs[j][6], bufs[j][7])
            pltpu.make_async_copy(a_hbm.at[isoj], av, sa).wait()
            pltpu.make_async_copy(u_hbm.at[iduj], uv, su).wait()

        def work(j):
            av, uv, idj = bufs[j][0], bufs[j][1], bufs[j][3]
            idcj, ssj = bufs[j][9], bufs[j][10]

            def row(i, c2):
                for rr in range(4):
                    for jj in range(Hc // 16):
                        sl = pl.ds(jj * 16, 16)
                        av[i * 4 + rr, sl] = jnp.maximum(
                            av[i * 4 + rr, sl], uv[i * 4 + rr, sl])
                return c2
            lax.fori_loop(0, EB // 4, row, 0)
            for jj in range(EB // 16):
                sl = pl.ds(jj * 16, 16)
                idcj[sl] = idj[sl]
            pltpu.make_async_copy(av, acc.at[idcj], ssj).start(add=True)

        def scat_wait(j):
            av, idcj, ssj = bufs[j][0], bufs[j][9], bufs[j][10]
            pltpu.make_async_copy(av, acc.at[idcj], ssj).wait()

        def chunk(c, carry):
            coff = c * NP

            def zrow(i, c2):
                for jj in range(Hc // 16):
                    u0[i, pl.ds(jj * 16, 16)] = jnp.zeros((16,), jnp.float32)
                return c2
            lax.fori_loop(0, EB, zrow, 0)

            def zacc(i, c2):
                pltpu.sync_copy(u0, acc.at[pl.ds(sid * rows_s + i * EB, EB)])
                return c2
            lax.fori_loop(0, rows_s // EB, zacc, 0)
            plsc.subcore_barrier()

            icp(0, 0)
            icp(1, 1)
            icp_wait(0, 0)
            off(0, coff)
            gath(0)

            def pair(g, c2):
                for j in range(2):
                    b = 2 * g + j
                    nj = 1 - j

                    @pl.when(b + 1 < nb_w)
                    def _():
                        @pl.when(b >= 1)
                        def _():
                            scat_wait(nj)
                        icp_wait(b + 1, nj)
                        off(nj, coff)
                        gath(nj)
                    gath_wait(j)
                    work(j)

                    @pl.when(b + 2 < nb_w)
                    def _():
                        icp(b + 2, j)
                return c2
            lax.fori_loop(0, nb_w // 2, pair, 0)
            scat_wait(0)
            scat_wait(1)
            plsc.subcore_barrier()

            def wout(i, c2):
                r0 = sid * rows_s + i * ZR
                pltpu.sync_copy(acc.at[pl.ds(r0, ZR)],
                                out_hbm.at[pl.ds((2 * c + cid) * NP + r0, ZR)])
                return c2
            lax.fori_loop(0, rows_s // ZR, wout, 0)
            return carry
        lax.fori_loop(0, Hch, chunk, 0)

    return k(A_cm, U_cm, src, dst)


def _sc_count(dst):
    Hc = 128
    nb_w = EP // EB // NW
    rows_s = NP // 16
    ZR = 32

    mesh = plsc.VectorSubcoreMesh(core_axis_name="c", subcore_axis_name="s")

    @functools.partial(
        pl.kernel, mesh=mesh,
        out_type=jax.ShapeDtypeStruct((2 * NP, Hc), jnp.float32),
        scratch_types=[
            pltpu.VMEM((nb_w, EB), jnp.int32),
            pltpu.VMEM((EB,), jnp.int32),
            pltpu.VMEM((EB, Hc), jnp.float32),
            pltpu.VMEM((ZR, Hc), jnp.float32),
            pltpu.VMEM_SHARED((NP, Hc), jnp.float32),
        ],
    )
    def k(dst_hbm, out_hbm, idst, idd, ones_v, zb, acc):
        cid = lax.axis_index("c")
        sid = lax.axis_index("s")
        wid = sid * 2 + cid

        def fill(i, carry):
            for jj in range(Hc // 16):
                zb[i, pl.ds(jj * 16, 16)] = jnp.zeros((16,), jnp.float32)
            return carry
        lax.fori_loop(0, ZR, fill, 0)

        def fill1(i, carry):
            for jj in range(Hc // 16):
                ones_v[i, pl.ds(jj * 16, 16)] = jnp.ones((16,), jnp.float32)
            return carry
        lax.fori_loop(0, EB, fill1, 0)

        def zacc(i, carry):
            pltpu.sync_copy(zb, acc.at[pl.ds(sid * rows_s + i * ZR, ZR)])
            return carry
        lax.fori_loop(0, rows_s // ZR, zacc, 0)
        pltpu.sync_copy(dst_hbm.at[pl.ds(wid * nb_w, nb_w)], idst)
        plsc.subcore_barrier()

        def blk(b, carry):
            for jj in range(EB // 16):
                sl = pl.ds(jj * 16, 16)
                idd[sl] = idst[b, sl]
            pltpu.sync_copy(ones_v, acc.at[idd], add=True)
            return carry
        lax.fori_loop(0, nb_w, blk, 0)
        plsc.subcore_barrier()

        def wout(i, carry):
            r0 = sid * rows_s + i * ZR
            pltpu.sync_copy(acc.at[pl.ds(r0, ZR)],
                            out_hbm.at[pl.ds(cid * NP + r0, ZR)])
            return carry
        lax.fori_loop(0, rows_s // ZR, wout, 0)

    return k(dst)


def _mm_body(x_ref, w_ref, b_ref, o_ref, acc_ref, *, nk, relu):
    @pl.when(pl.program_id(2) == 0)
    def _():
        acc_ref[...] = jnp.zeros_like(acc_ref)
    acc_ref[...] += jnp.dot(x_ref[...], w_ref[...],
                            preferred_element_type=jnp.float32)
    @pl.when(pl.program_id(2) == nk - 1)
    def _():
        r = acc_ref[...] + b_ref[...][0:1, :]
        o_ref[...] = jnp.maximum(r, 0.0) if relu else r


def _mm(x, w, b, relu, bm=256, bn=256, bk=128):
    M, K = x.shape
    Nn = w.shape[1]
    bn = min(bn, Nn)
    b8 = jnp.tile(b[None, :], (8, 1))
    nk = K // bk
    return pl.pallas_call(
        functools.partial(_mm_body, nk=nk, relu=relu),
        grid=(M // bm, Nn // bn, nk),
        in_specs=[
            pl.BlockSpec((bm, bk), lambda i, j, k: (i, k)),
            pl.BlockSpec((bk, bn), lambda i, j, k: (k, j)),
            pl.BlockSpec((8, bn), lambda i, j, k: (0, j)),
        ],
        out_specs=pl.BlockSpec((bm, bn), lambda i, j, k: (i, j)),
        out_shape=jax.ShapeDtypeStruct((M, Nn), jnp.float32),
        scratch_shapes=[pltpu.VMEM((bm, bn), jnp.float32)],
    )(x, w, b8)


def _stage1_body(x_ref, wh_ref, pos_ref, wp_ref, b_ref, a_ref, u_ref,
                 acc_ref, *, nk):
    @pl.when(pl.program_id(2) == 0)
    def _():
        acc_ref[...] = jnp.zeros_like(acc_ref)
    acc_ref[...] += jnp.dot(x_ref[...], wh_ref[...],
                            preferred_element_type=jnp.float32)
    @pl.when(pl.program_id(2) == nk - 1)
    def _():
        pu = jnp.dot(pos_ref[...], wp_ref[...],
                     preferred_element_type=jnp.float32)
        u_ref[...] = pu
        a_ref[...] = acc_ref[...] + pu + b_ref[...][0:1, :]


def _stage1(x, posp, wh, wp, ba, bm=256, bk=128):
    M, K = x.shape
    H = wh.shape[1]
    bn = 128
    npb = M // bm
    b8 = jnp.tile(ba[None, :], (8, 1))
    nk = K // bk
    return pl.pallas_call(
        functools.partial(_stage1_body, nk=nk),
        grid=(M // bm, H // bn, nk),
        in_specs=[
            pl.BlockSpec((bm, bk), lambda i, j, k: (i, k)),
            pl.BlockSpec((bk, bn), lambda i, j, k: (k, j)),
            pl.BlockSpec((bm, 8), lambda i, j, k: (i, 0)),
            pl.BlockSpec((8, bn), lambda i, j, k: (0, j)),
            pl.BlockSpec((8, bn), lambda i, j, k: (0, j)),
        ],
        out_specs=[
            pl.BlockSpec((bm, bn), lambda i, j, k: (j * npb + i, 0)),
            pl.BlockSpec((bm, bn), lambda i, j, k: (j * npb + i, 0)),
        ],
        out_shape=[
            jax.ShapeDtypeStruct((H // bn * M, bn), jnp.float32),
            jax.ShapeDtypeStruct((H // bn * M, bn), jnp.float32),
        ],
        scratch_shapes=[pltpu.VMEM((bm, bn), jnp.float32)],
    )(x, wh, posp, wp, b8)


def _stage2_body(s0_ref, s1_ref, u_ref, c0_ref, c1_ref, w_ref, b_ref, o_ref,
                 acc_ref, *, nk, relu):
    @pl.when(pl.program_id(2) == 0)
    def _():
        acc_ref[...] = jnp.zeros_like(acc_ref)
    c = c0_ref[...][:, 0:1] + c1_ref[...][:, 0:1]
    inv = 1.0 / jnp.maximum(c, 1.0)
    m = (s0_ref[...] + s1_ref[...] - c * u_ref[...]) * inv
    acc_ref[...] += jnp.dot(m, w_ref[...], preferred_element_type=jnp.float32)
    @pl.when(pl.program_id(2) == nk - 1)
    def _():
        mask = (c > 0.0).astype(jnp.float32)
        r = acc_ref[...] + mask * b_ref[...][0:1, :]
        o_ref[...] = jnp.maximum(r, 0.0) if relu else r


def _stage2(sc_out, u_cm, c0, c1, wb, bb, relu, bm=256, bn=256):
    H = wb.shape[0]
    O = wb.shape[1]
    bn = min(bn, O)
    bk = 128
    npb = NP // bm
    b8 = jnp.tile(bb[None, :], (8, 1))
    nk = H // bk
    return pl.pallas_call(
        functools.partial(_stage2_body, nk=nk, relu=relu),
        grid=(NP // bm, O // bn, nk),
        in_specs=[
            pl.BlockSpec((bm, bk), lambda i, j, k: ((2 * k) * npb + i, 0)),
            pl.BlockSpec((bm, bk), lambda i, j, k: ((2 * k + 1) * npb + i, 0)),
            pl.BlockSpec((bm, bk), lambda i, j, k: (k * npb + i, 0)),
            pl.BlockSpec((bm, 16), lambda i, j, k: (i, 0)),
            pl.BlockSpec((bm, 16), lambda i, j, k: (i, 0)),
            pl.BlockSpec((bk, bn), lambda i, j, k: (k, j)),
            pl.BlockSpec((8, bn), lambda i, j, k: (0, j)),
        ],
        out_specs=pl.BlockSpec((bm, bn), lambda i, j, k: (i, j)),
        out_shape=jax.ShapeDtypeStruct((NP, O), jnp.float32),
        scratch_shapes=[pltpu.VMEM((bm, bn), jnp.float32)],
    )(sc_out, sc_out, u_cm, c0, c1, wb, b8)


def _conv_sc(x, posp, src_p, dst_p, c0, c1, wa, ba, wb, bb):
    cin = x.shape[1]
    H = wa.shape[1]
    wh = wa[:cin]
    wp = jnp.zeros((8, H), jnp.float32).at[:2].set(wa[cin:cin + 2])
    A_cm, U_cm = _stage1(x, posp, wh, wp, ba)
    out = _sc_edge_agg(A_cm, U_cm, src_p, dst_p, H // 128)
    return _stage2(out, U_cm, c0, c1, wb, bb, relu=True)


def kernel(h, pos, edge_index, params):
    N = h.shape[0]
    p = params
    src = edge_index[0]
    dst = edge_index[1]
    pad_idx = jnp.full((EP - src.shape[0],), N, jnp.int32)
    src_p = jnp.concatenate([src, pad_idx])
    dst_p = jnp.concatenate([dst, pad_idx])

    x = jnp.zeros((NP, h.shape[1]), jnp.float32).at[:N].set(h)
    posp = jnp.zeros((NP, 8), jnp.float32).at[:N, :2].set(pos)

    cnt = _sc_count(dst_p.reshape(EP // EB, EB))
    c0, c1 = cnt[:NP, :16], cnt[NP:, :16]

    x = _conv_sc(x, posp, src_p, dst_p, c0, c1, p["W0"], p["b0"], p["W1"], p["b1"])
    x = _conv_sc(x, posp, src_p, dst_p, c0, c1, p["W2"], p["b2"], p["W3"], p["b3"])
    x = _conv_sc(x, posp, src_p, dst_p, c0, c1, p["W4"], p["b4"], p["W5"], p["b5"])
    x = _conv_sc(x, posp, src_p, dst_p, c0, c1, p["W6"], p["b6"], p["W7"], p["b7"])

    x = _mm(x, p["W8"], p["b8"], relu=True)
    x = _mm(x, p["W9"], p["b9"], relu=True)
    x = _mm(x, p["W10"], p["b10"], relu=False)
    return x[:N]


# speedup vs baseline: 1.0733x; 1.0733x over previous
import functools

import jax
import jax.numpy as jnp
from jax import lax
from jax.experimental import pallas as pl
from jax.experimental.pallas import tpu as pltpu
from jax.experimental.pallas import tpu_sc as plsc

NP = 10240
EB = 64
NW = 32
EP = 163840


def _sc_edge_agg(A_cm, U_cm, src, dst, Hch):
    Hc = 128
    nb_w = EP // EB // NW
    rows_s = NP // 16
    ZR = 64

    mesh = plsc.VectorSubcoreMesh(core_axis_name="c", subcore_axis_name="s")

    @functools.partial(
        pl.kernel, mesh=mesh,
        out_type=jax.ShapeDtypeStruct((Hch * 2 * NP, Hc), jnp.float32),
        scratch_types=[
            pltpu.VMEM((EB, Hc), jnp.float32),
            pltpu.VMEM((EB, Hc), jnp.float32),
            pltpu.VMEM((EB, Hc), jnp.float32),
            pltpu.VMEM((EB, Hc), jnp.float32),
            pltpu.VMEM((EB,), jnp.int32),
            pltpu.VMEM((EB,), jnp.int32),
            pltpu.VMEM((EB,), jnp.int32),
            pltpu.VMEM((EB,), jnp.int32),
            pltpu.VMEM((EB,), jnp.int32),
            pltpu.VMEM((EB,), jnp.int32),
            pltpu.VMEM((EB,), jnp.int32),
            pltpu.VMEM((EB,), jnp.int32),
            pltpu.VMEM((EB,), jnp.int32),
            pltpu.VMEM((EB,), jnp.int32),
            pltpu.VMEM_SHARED((NP, Hc), jnp.float32),
            pltpu.SemaphoreType.DMA,
            pltpu.SemaphoreType.DMA,
            pltpu.SemaphoreType.DMA,
            pltpu.SemaphoreType.DMA,
            pltpu.SemaphoreType.DMA,
            pltpu.SemaphoreType.DMA,
            pltpu.SemaphoreType.DMA,
            pltpu.SemaphoreType.DMA,
        ],
    )
    def k(a_hbm, u_hbm, src_hbm, dst_hbm, out_hbm,
          a0, u0, a1, u1, is0, id0, iso0, idu0, is1, id1, iso1, idu1,
          idc0, idc1, acc, sa0, su0, sa1, su1, si0, si1, ss0, ss1):
        cid = lax.axis_index("c")
        sid = lax.axis_index("s")
        wid = sid * 2 + cid
        bufs = ((a0, u0, is0, id0, iso0, idu0, sa0, su0, si0, idc0, ss0),
                (a1, u1, is1, id1, iso1, idu1, sa1, su1, si1, idc1, ss1))

        def icp(b, j):
            isj, idj, si = bufs[j][2], bufs[j][3], bufs[j][8]
            base = (wid * nb_w + b) * EB
            pltpu.make_async_copy(src_hbm.at[pl.ds(base, EB)], isj, si).start()
            pltpu.make_async_copy(dst_hbm.at[pl.ds(base, EB)], idj, si).start()

        def icp_wait(b, j):
            isj, idj, si = bufs[j][2], bufs[j][3], bufs[j][8]
            base = (wid * nb_w + b) * EB
            pltpu.make_async_copy(src_hbm.at[pl.ds(base, EB)], isj, si).wait()
            pltpu.make_async_copy(dst_hbm.at[pl.ds(base, EB)], idj, si).wait()

        def off(j, coff):
            isj, idj, isoj, iduj = bufs[j][2], bufs[j][3], bufs[j][4], bufs[j][5]
            for jj in range(EB // 16):
                sl = pl.ds(jj * 16, 16)
                isoj[sl] = isj[sl] + coff
                iduj[sl] = idj[sl] + coff

        def gath(j):
            av, uv, isoj, iduj, sa, su = (bufs[j][0], bufs[j][1], bufs[j][4],
                                          bufs[j][5], bufs[j][6], bufs[j][7])
            pltpu.make_async_copy(a_hbm.at[isoj], av, sa).start()
            pltpu.make_async_copy(u_hbm.at[iduj], uv, su).start()

        def gath_wait(j):
            av, uv, isoj, iduj, sa, su = (bufs[j][0], bufs[j][1], bufs[j][4],
                                          bufs[j][5], bufs[j][6], bufs[j][7])
            pltpu.make_async_copy(a_hbm.at[isoj], av, sa).wait()
            pltpu.make_async_copy(u_hbm.at[iduj], uv, su).wait()

        def work(j):
            av, uv, idj = bufs[j][0], bufs[j][1], bufs[j][3]
            idcj, ssj = bufs[j][9], bufs[j][10]

            def row(i, c2):
                for rr in range(4):
                    for jj in range(Hc // 16):
                        sl = pl.ds(jj * 16, 16)
                        av[i * 4 + rr, sl] = jnp.maximum(
                            av[i * 4 + rr, sl] - uv[i * 4 + rr, sl], 0.0)
                return c2
            lax.fori_loop(0, EB // 4, row, 0)
            for jj in range(EB // 16):
                sl = pl.ds(jj * 16, 16)
                idcj[sl] = idj[sl]
            pltpu.make_async_copy(av, acc.at[idcj], ssj).start(add=True)

        def scat_wait(j):
            av, idcj, ssj = bufs[j][0], bufs[j][9], bufs[j][10]
            pltpu.make_async_copy(av, acc.at[idcj], ssj).wait()

        def chunk(c, carry):
            coff = c * NP

            def zrow(i, c2):
                for jj in range(Hc // 16):
                    u0[i, pl.ds(jj * 16, 16)] = jnp.zeros((16,), jnp.float32)
                return c2
            lax.fori_loop(0, EB, zrow, 0)

            def zacc(i, c2):
                pltpu.sync_copy(u0, acc.at[pl.ds(sid * rows_s + i * EB, EB)])
                return c2
            lax.fori_loop(0, rows_s // EB, zacc, 0)
            plsc.subcore_barrier()

            icp(0, 0)
            icp(1, 1)
            icp_wait(0, 0)
            off(0, coff)
            gath(0)

            def pair(g, c2):
                for j in range(2):
                    b = 2 * g + j
                    nj = 1 - j

                    @pl.when(b + 1 < nb_w)
                    def _():
                        @pl.when(b >= 1)
                        def _():
                            scat_wait(nj)
                        icp_wait(b + 1, nj)
                        off(nj, coff)
                        gath(nj)
                    gath_wait(j)
                    work(j)

                    @pl.when(b + 2 < nb_w)
                    def _():
                        icp(b + 2, j)
                return c2
            lax.fori_loop(0, nb_w // 2, pair, 0)
            scat_wait(0)
            scat_wait(1)
            plsc.subcore_barrier()

            def wout(i, c2):
                r0 = sid * rows_s + i * ZR
                pltpu.sync_copy(acc.at[pl.ds(r0, ZR)],
                                out_hbm.at[pl.ds((2 * c + cid) * NP + r0, ZR)])
                return c2
            lax.fori_loop(0, rows_s // ZR, wout, 0)
            return carry
        lax.fori_loop(0, Hch, chunk, 0)

    return k(A_cm, U_cm, src, dst)


def _sc_count(dst):
    Hc = 128
    nb_w = EP // EB // NW
    rows_s = NP // 16
    ZR = 32

    mesh = plsc.VectorSubcoreMesh(core_axis_name="c", subcore_axis_name="s")

    @functools.partial(
        pl.kernel, mesh=mesh,
        out_type=jax.ShapeDtypeStruct((2 * NP, Hc), jnp.float32),
        scratch_types=[
            pltpu.VMEM((nb_w, EB), jnp.int32),
            pltpu.VMEM((EB,), jnp.int32),
            pltpu.VMEM((EB, Hc), jnp.float32),
            pltpu.VMEM((ZR, Hc), jnp.float32),
            pltpu.VMEM_SHARED((NP, Hc), jnp.float32),
        ],
    )
    def k(dst_hbm, out_hbm, idst, idd, ones_v, zb, acc):
        cid = lax.axis_index("c")
        sid = lax.axis_index("s")
        wid = sid * 2 + cid

        def fill(i, carry):
            for jj in range(Hc // 16):
                zb[i, pl.ds(jj * 16, 16)] = jnp.zeros((16,), jnp.float32)
            return carry
        lax.fori_loop(0, ZR, fill, 0)

        def fill1(i, carry):
            for jj in range(Hc // 16):
                ones_v[i, pl.ds(jj * 16, 16)] = jnp.ones((16,), jnp.float32)
            return carry
        lax.fori_loop(0, EB, fill1, 0)

        def zacc(i, carry):
            pltpu.sync_copy(zb, acc.at[pl.ds(sid * rows_s + i * ZR, ZR)])
            return carry
        lax.fori_loop(0, rows_s // ZR, zacc, 0)
        pltpu.sync_copy(dst_hbm.at[pl.ds(wid * nb_w, nb_w)], idst)
        plsc.subcore_barrier()

        def blk(b, carry):
            for jj in range(EB // 16):
                sl = pl.ds(jj * 16, 16)
                idd[sl] = idst[b, sl]
            pltpu.sync_copy(ones_v, acc.at[idd], add=True)
            return carry
        lax.fori_loop(0, nb_w, blk, 0)
        plsc.subcore_barrier()

        def wout(i, carry):
            r0 = sid * rows_s + i * ZR
            pltpu.sync_copy(acc.at[pl.ds(r0, ZR)],
                            out_hbm.at[pl.ds(cid * NP + r0, ZR)])
            return carry
        lax.fori_loop(0, rows_s // ZR, wout, 0)

    return k(dst)


def _mm_body(x_ref, w_ref, b_ref, o_ref, acc_ref, *, nk, relu):
    @pl.when(pl.program_id(2) == 0)
    def _():
        acc_ref[...] = jnp.zeros_like(acc_ref)
    acc_ref[...] += jnp.dot(x_ref[...], w_ref[...],
                            preferred_element_type=jnp.float32)
    @pl.when(pl.program_id(2) == nk - 1)
    def _():
        r = acc_ref[...] + b_ref[...][0:1, :]
        o_ref[...] = jnp.maximum(r, 0.0) if relu else r


def _mm(x, w, b, relu, bm=256, bn=256, bk=128):
    M, K = x.shape
    Nn = w.shape[1]
    bn = min(bn, Nn)
    b8 = jnp.tile(b[None, :], (8, 1))
    nk = K // bk
    return pl.pallas_call(
        functools.partial(_mm_body, nk=nk, relu=relu),
        grid=(M // bm, Nn // bn, nk),
        in_specs=[
            pl.BlockSpec((bm, bk), lambda i, j, k: (i, k)),
            pl.BlockSpec((bk, bn), lambda i, j, k: (k, j)),
            pl.BlockSpec((8, bn), lambda i, j, k: (0, j)),
        ],
        out_specs=pl.BlockSpec((bm, bn), lambda i, j, k: (i, j)),
        out_shape=jax.ShapeDtypeStruct((M, Nn), jnp.float32),
        scratch_shapes=[pltpu.VMEM((bm, bn), jnp.float32)],
    )(x, w, b8)


def _stage1_body(x_ref, wh_ref, pos_ref, wp_ref, b_ref, a_ref, u_ref,
                 acc_ref, *, nk):
    @pl.when(pl.program_id(2) == 0)
    def _():
        acc_ref[...] = jnp.zeros_like(acc_ref)
    acc_ref[...] += jnp.dot(x_ref[...], wh_ref[...],
                            preferred_element_type=jnp.float32)
    @pl.when(pl.program_id(2) == nk - 1)
    def _():
        pu = jnp.dot(pos_ref[...], wp_ref[...],
                     preferred_element_type=jnp.float32)
        u_ref[...] = pu
        a_ref[...] = acc_ref[...] + pu + b_ref[...][0:1, :]


def _stage1(x, posp, wh, wp, ba, bm=256, bn=256, bk=128):
    M, K = x.shape
    H = wh.shape[1]
    bn = min(bn, H)
    b8 = jnp.tile(ba[None, :], (8, 1))
    nk = K // bk
    return pl.pallas_call(
        functools.partial(_stage1_body, nk=nk),
        grid=(M // bm, H // bn, nk),
        in_specs=[
            pl.BlockSpec((bm, bk), lambda i, j, k: (i, k)),
            pl.BlockSpec((bk, bn), lambda i, j, k: (k, j)),
            pl.BlockSpec((bm, 8), lambda i, j, k: (i, 0)),
            pl.BlockSpec((8, bn), lambda i, j, k: (0, j)),
            pl.BlockSpec((8, bn), lambda i, j, k: (0, j)),
        ],
        out_specs=[
            pl.BlockSpec((bm, bn), lambda i, j, k: (i, j)),
            pl.BlockSpec((bm, bn), lambda i, j, k: (i, j)),
        ],
        out_shape=[
            jax.ShapeDtypeStruct((M, H), jnp.float32),
            jax.ShapeDtypeStruct((M, H), jnp.float32),
        ],
        scratch_shapes=[pltpu.VMEM((bm, bn), jnp.float32)],
    )(x, wh, posp, wp, b8)


def _stage2_body(s0_ref, s1_ref, c0_ref, c1_ref, w_ref, b_ref, o_ref,
                 acc_ref, *, nk, relu):
    @pl.when(pl.program_id(2) == 0)
    def _():
        acc_ref[...] = jnp.zeros_like(acc_ref)
    c = c0_ref[...][:, 0:1] + c1_ref[...][:, 0:1]
    inv = 1.0 / jnp.maximum(c, 1.0)
    m = (s0_ref[...] + s1_ref[...]) * inv
    acc_ref[...] += jnp.dot(m, w_ref[...], preferred_element_type=jnp.float32)
    @pl.when(pl.program_id(2) == nk - 1)
    def _():
        mask = (c > 0.0).astype(jnp.float32)
        r = acc_ref[...] + mask * b_ref[...][0:1, :]
        o_ref[...] = jnp.maximum(r, 0.0) if relu else r


def _stage2(s0, s1, c0, c1, wb, bb, relu, bm=256, bn=256, bk=128):
    M, H = s0.shape
    O = wb.shape[1]
    bn = min(bn, O)
    b8 = jnp.tile(bb[None, :], (8, 1))
    nk = H // bk
    return pl.pallas_call(
        functools.partial(_stage2_body, nk=nk, relu=relu),
        grid=(M // bm, O // bn, nk),
        in_specs=[
            pl.BlockSpec((bm, bk), lambda i, j, k: (i, k)),
            pl.BlockSpec((bm, bk), lambda i, j, k: (i, k)),
            pl.BlockSpec((bm, 16), lambda i, j, k: (i, 0)),
            pl.BlockSpec((bm, 16), lambda i, j, k: (i, 0)),
            pl.BlockSpec((bk, bn), lambda i, j, k: (k, j)),
            pl.BlockSpec((8, bn), lambda i, j, k: (0, j)),
        ],
        out_specs=pl.BlockSpec((bm, bn), lambda i, j, k: (i, j)),
        out_shape=jax.ShapeDtypeStruct((M, O), jnp.float32),
        scratch_shapes=[pltpu.VMEM((bm, bn), jnp.float32)],
    )(s0, s1, c0, c1, wb, b8)


def _conv_sc(x, posp, src_p, dst_p, c0, c1, wa, ba, wb, bb):
    cin = x.shape[1]
    H = wa.shape[1]
    wh = wa[:cin]
    wp = jnp.zeros((8, H), jnp.float32).at[:2].set(wa[cin:cin + 2])
    A, U = _stage1(x, posp, wh, wp, ba)
    parts = []
    for c in range(H // 128):
        sl = slice(c * 128, (c + 1) * 128)
        parts.append(_sc_edge_agg(A[:, sl], U[:, sl], src_p, dst_p, 1))
    s0 = jnp.concatenate([p[:NP] for p in parts], axis=1)
    s1 = jnp.concatenate([p[NP:] for p in parts], axis=1)
    return _stage2(s0, s1, c0, c1, wb, bb, relu=True)


def kernel(h, pos, edge_index, params):
    N = h.shape[0]
    p = params
    src = edge_index[0]
    dst = edge_index[1]
    pad_idx = jnp.full((EP - src.shape[0],), N, jnp.int32)
    src_p = jnp.concatenate([src, pad_idx])
    dst_p = jnp.concatenate([dst, pad_idx])

    x = jnp.zeros((NP, h.shape[1]), jnp.float32).at[:N].set(h)
    posp = jnp.zeros((NP, 8), jnp.float32).at[:N, :2].set(pos)

    cnt = _sc_count(dst_p.reshape(EP // EB, EB))
    c0, c1 = cnt[:NP, :16], cnt[NP:, :16]

    x = _conv_sc(x, posp, src_p, dst_p, c0, c1, p["W0"], p["b0"], p["W1"], p["b1"])
    x = _conv_sc(x, posp, src_p, dst_p, c0, c1, p["W2"], p["b2"], p["W3"], p["b3"])
    x = _conv_sc(x, posp, src_p, dst_p, c0, c1, p["W4"], p["b4"], p["W5"], p["b5"])
    x = _conv_sc(x, posp, src_p, dst_p, c0, c1, p["W6"], p["b6"], p["W7"], p["b7"])

    x = _mm(x, p["W8"], p["b8"], relu=True)
    x = _mm(x, p["W9"], p["b9"], relu=True)
    x = _mm(x, p["W10"], p["b10"], relu=False)
    return x[:N]
